# Initial kernel scaffold; baseline (speedup 1.0000x reference)
import jax
import jax.numpy as jnp
from jax.experimental import pallas as pl


def kernel(nodes, edges, edge_types, atoms, atom_edges, params, bias, ho_params, ho_bias):
    raise NotImplementedError("write your pallas kernel here")


# speedup vs baseline: 44.7514x; 44.7514x over previous
import functools

import jax
import jax.numpy as jnp
from jax import lax
from jax.experimental import pallas as pl
from jax.experimental.pallas import tpu as pltpu
from jax.experimental.pallas import tpu_sc as plsc

N = 2048
E = 8192
ORDER = 3
LATENT = 64
RANK = 64
NUM_PARAMS = 4

NC = 2
NS = 16
NW = NC * NS
R_TOTAL = ORDER * E
CHUNK = 128
R_PER_W = R_TOTAL // NW
G_CHUNKS = R_PER_W // CHUNK
R_PER_S = R_TOTAL // NS
S_CHUNKS = R_PER_S // CHUNK

_SC_PARAMS = pltpu.CompilerParams(use_tc_tiling_on_sc=False)


def _tc_pre_body(nodes_ref, w1_ref, b1_ref, out_ref):
    x = nodes_ref[...]
    for t in range(NUM_PARAMS):
        acc = lax.dot_general(
            x, w1_ref[t], (((1,), (0,)), ((), ())),
            preferred_element_type=jnp.float32,
        )
        b = b1_ref[...]
        out_ref[t] = jnp.maximum(acc + lax.slice(b, (t, 0), (t + 1, RANK)), 0.0)


def _tc_pre(nodes, w1, b1, interpret=False):
    return pl.pallas_call(
        _tc_pre_body,
        out_shape=jax.ShapeDtypeStruct((NUM_PARAMS, N, RANK), jnp.float32),
        interpret=interpret,
    )(nodes, w1, b1)


def _sc_gather(table, idx3d):
    mesh = plsc.VectorSubcoreMesh(core_axis_name="c", subcore_axis_name="s")

    @functools.partial(
        pl.kernel,
        out_type=jax.ShapeDtypeStruct((R_TOTAL, RANK), jnp.float32),
        mesh=mesh,
        scratch_types=[
            pltpu.VMEM((G_CHUNKS, CHUNK), jnp.int32),
            pltpu.VMEM((R_PER_W, RANK), jnp.float32),
            [pltpu.SemaphoreType.DMA] * G_CHUNKS,
            pltpu.SemaphoreType.DMA,
        ],
        compiler_params=_SC_PARAMS,
    )
    def gk(table_hbm, idx_hbm, out_hbm, idx_v, rows_v, sems, sem_wb):
        wid = lax.axis_index("s") * NC + lax.axis_index("c")
        base = wid * R_PER_W
        pltpu.sync_copy(idx_hbm.at[wid], idx_v)
        copies = []
        for j in range(G_CHUNKS):
            copies.append(
                pltpu.async_copy(
                    table_hbm.at[idx_v.at[j]],
                    rows_v.at[pl.ds(j * CHUNK, CHUNK)],
                    sems[j],
                )
            )
        wbs = []
        for j in range(G_CHUNKS):
            copies[j].wait()
            wbs.append(
                pltpu.async_copy(
                    rows_v.at[pl.ds(j * CHUNK, CHUNK)],
                    out_hbm.at[pl.ds(base + j * CHUNK, CHUNK)],
                    sem_wb,
                )
            )
        for w in wbs:
            w.wait()

    return gk(table, idx3d)


def _sc_scatter_add(msgs, idx3d, zeros):
    mesh = plsc.VectorSubcoreMesh(core_axis_name="c", subcore_axis_name="s")
    half = N // NC

    @functools.partial(
        pl.kernel,
        out_type=jax.ShapeDtypeStruct((N, LATENT), jnp.float32),
        mesh=mesh,
        scratch_types=[
            pltpu.VMEM((S_CHUNKS, CHUNK), jnp.int32),
            pltpu.VMEM((R_PER_S, LATENT), jnp.float32),
            pltpu.VMEM_SHARED((N, LATENT), jnp.float32),
            pltpu.SemaphoreType.DMA,
            pltpu.SemaphoreType.DMA,
            pltpu.SemaphoreType.DMA,
        ],
        compiler_params=_SC_PARAMS,
    )
    def sk(msgs_hbm, idx_hbm, zeros_hbm, out_hbm, idx_v, msg_v, acc, sem_z, sem_in, sem_sc):
        cid = lax.axis_index("c")
        sid = lax.axis_index("s")
        rows_per_tile = N // NS
        zcp = pltpu.async_copy(
            zeros_hbm.at[pl.ds(sid * rows_per_tile, rows_per_tile)],
            acc.at[pl.ds(sid * rows_per_tile, rows_per_tile)],
            sem_z,
        )
        pltpu.sync_copy(idx_hbm.at[sid], idx_v)
        mcp = pltpu.async_copy(
            msgs_hbm.at[pl.ds(sid * R_PER_S, R_PER_S)], msg_v, sem_in
        )
        zcp.wait()
        plsc.subcore_barrier()
        mcp.wait()
        adds = []
        for j in range(S_CHUNKS):
            adds.append(
                pltpu.async_copy(
                    msg_v.at[pl.ds(j * CHUNK, CHUNK)],
                    acc.at[idx_v.at[j]],
                    sem_sc,
                    add=True,
                )
            )
        for a in adds:
            a.wait()
        plsc.subcore_barrier()
        out_rows = half // NS
        start = cid * half + sid * out_rows
        pltpu.sync_copy(
            acc.at[pl.ds(start, out_rows)], out_hbm.at[pl.ds(start, out_rows)]
        )

    return sk(msgs, idx3d, zeros)


BE = 2048


def _tc_dense_body(tr_ref, ids_ref, w2_ref, b2_ref, out_ref):
    ids = ids_ref[...]
    masks = []
    for i in range(ORDER):
        idv = lax.slice(ids, (0, i), (BE, i + 1))
        masks.append([idv == t for t in range(NUM_PARAMS)])
    transforms = [tr_ref[i] for i in range(ORDER)]
    facts = [
        transforms[1] * transforms[2],
        transforms[0] * transforms[2],
        transforms[0] * transforms[1],
    ]
    for i in range(ORDER):
        fcat = jnp.concatenate(
            [jnp.where(masks[i][t], facts[i], 0.0) for t in range(NUM_PARAMS)],
            axis=1,
        )
        msg = lax.dot_general(
            fcat, w2_ref[i], (((1,), (0,)), ((), ())),
            preferred_element_type=jnp.float32,
        )
        b2 = b2_ref[i]
        for t in range(NUM_PARAMS):
            msg = msg + jnp.where(masks[i][t], lax.slice(b2, (t, 0), (t + 1, LATENT)), 0.0)
        out_ref[i] = msg


def _tc_dense(transforms, ids, w2, b2, interpret=False):
    nb = E // BE
    return pl.pallas_call(
        _tc_dense_body,
        out_shape=jax.ShapeDtypeStruct((ORDER, E, LATENT), jnp.float32),
        grid=(nb,),
        in_specs=[
            pl.BlockSpec((ORDER, BE, RANK), lambda b: (0, b, 0)),
            pl.BlockSpec((BE, ORDER), lambda b: (b, 0)),
            pl.BlockSpec((ORDER, NUM_PARAMS * RANK, LATENT), lambda b: (0, 0, 0)),
            pl.BlockSpec((ORDER, NUM_PARAMS, LATENT), lambda b: (0, 0, 0)),
        ],
        out_specs=pl.BlockSpec((ORDER, BE, LATENT), lambda b: (0, b, 0)),
        interpret=interpret,
    )(transforms, ids, w2, b2)


def kernel(nodes, edges, edge_types, atoms, atom_edges, params, bias, ho_params, ho_bias):
    del atoms, atom_edges
    b1 = bias[:, 0, :]
    table = _tc_pre(nodes, params, b1)
    gidx = (edge_types.T.astype(jnp.int32) * N + edges.T.astype(jnp.int32))
    idx_g = gidx.reshape(NW, G_CHUNKS, CHUNK)
    idx_s = edges.T.reshape(NS, S_CHUNKS, CHUNK)
    tr = _sc_gather(table.reshape(NUM_PARAMS * N, RANK), idx_g)
    tr = tr.reshape(ORDER, E, RANK)
    w2 = ho_params.reshape(ORDER, NUM_PARAMS * RANK, LATENT)
    b2 = ho_bias[:, :, 0, :]
    msgs = _tc_dense(tr, edge_types, w2, b2)
    zeros = jnp.zeros((N, LATENT), jnp.float32)
    return _sc_scatter_add(msgs.reshape(R_TOTAL, LATENT), idx_s, zeros)


# speedup vs baseline: 45.3269x; 1.0129x over previous
import functools

import jax
import jax.numpy as jnp
from jax import lax
from jax.experimental import pallas as pl
from jax.experimental.pallas import tpu as pltpu
from jax.experimental.pallas import tpu_sc as plsc

N = 2048
E = 8192
ORDER = 3
LATENT = 64
RANK = 64
NUM_PARAMS = 4

NC = 2
NS = 16
NW = NC * NS
R_TOTAL = ORDER * E
CHUNK = 128
R_PER_W = R_TOTAL // NW
G_CHUNKS = R_PER_W // CHUNK
R_PER_S = R_TOTAL // NS
S_CHUNKS = R_PER_S // CHUNK

_SC_PARAMS = pltpu.CompilerParams(use_tc_tiling_on_sc=False)


def _tc_pre_body(nodes_ref, w1_ref, b1_ref, out_ref):
    x = nodes_ref[...]
    for t in range(NUM_PARAMS):
        acc = lax.dot_general(
            x, w1_ref[t], (((1,), (0,)), ((), ())),
            preferred_element_type=jnp.float32,
        )
        b = b1_ref[...]
        out_ref[pl.ds(t * N, N), :] = jnp.maximum(
            acc + lax.slice(b, (t, 0), (t + 1, RANK)), 0.0
        )


def _tc_pre(nodes, w1, b1, interpret=False):
    return pl.pallas_call(
        _tc_pre_body,
        out_shape=jax.ShapeDtypeStruct((NUM_PARAMS * N, RANK), jnp.float32),
        interpret=interpret,
    )(nodes, w1, b1)


def _sc_gather(table, idx3d):
    mesh = plsc.VectorSubcoreMesh(core_axis_name="c", subcore_axis_name="s")

    chunks_per_i = E // CHUNK

    @functools.partial(
        pl.kernel,
        out_type=jax.ShapeDtypeStruct((ORDER, E, RANK), jnp.float32),
        mesh=mesh,
        scratch_types=[
            pltpu.VMEM((G_CHUNKS, CHUNK), jnp.int32),
            pltpu.VMEM((R_PER_W, RANK), jnp.float32),
            [pltpu.SemaphoreType.DMA] * G_CHUNKS,
            pltpu.SemaphoreType.DMA,
        ],
        compiler_params=_SC_PARAMS,
    )
    def gk(table_hbm, idx_hbm, out_hbm, idx_v, rows_v, sems, sem_wb):
        wid = lax.axis_index("s") * NC + lax.axis_index("c")
        pltpu.sync_copy(idx_hbm.at[wid], idx_v)
        copies = []
        for j in range(G_CHUNKS):
            copies.append(
                pltpu.async_copy(
                    table_hbm.at[idx_v.at[j]],
                    rows_v.at[pl.ds(j * CHUNK, CHUNK)],
                    sems[j],
                )
            )
        wbs = []
        for j in range(G_CHUNKS):
            c = wid * G_CHUNKS + j
            ic = c // chunks_per_i
            eo = pl.multiple_of((c % chunks_per_i) * CHUNK, CHUNK)
            copies[j].wait()
            wbs.append(
                pltpu.async_copy(
                    rows_v.at[pl.ds(j * CHUNK, CHUNK)],
                    out_hbm.at[ic, pl.ds(eo, CHUNK)],
                    sem_wb,
                )
            )
        for w in wbs:
            w.wait()

    return gk(table, idx3d)


def _sc_scatter_add(msgs, idx3d, zeros):
    mesh = plsc.VectorSubcoreMesh(core_axis_name="c", subcore_axis_name="s")
    half = N // NC
    chunks_per_i = E // CHUNK

    @functools.partial(
        pl.kernel,
        out_type=jax.ShapeDtypeStruct((N, LATENT), jnp.float32),
        mesh=mesh,
        scratch_types=[
            pltpu.VMEM((S_CHUNKS, CHUNK), jnp.int32),
            pltpu.VMEM((R_PER_S, LATENT), jnp.float32),
            pltpu.VMEM_SHARED((N, LATENT), jnp.float32),
            pltpu.SemaphoreType.DMA,
            [pltpu.SemaphoreType.DMA] * S_CHUNKS,
            pltpu.SemaphoreType.DMA,
        ],
        compiler_params=_SC_PARAMS,
    )
    def sk(msgs_hbm, idx_hbm, zeros_hbm, out_hbm, idx_v, msg_v, acc, sem_z, sems, sem_sc):
        cid = lax.axis_index("c")
        sid = lax.axis_index("s")
        rows_per_tile = N // NS
        zcp = pltpu.async_copy(
            zeros_hbm.at[pl.ds(sid * rows_per_tile, rows_per_tile)],
            acc.at[pl.ds(sid * rows_per_tile, rows_per_tile)],
            sem_z,
        )
        pltpu.sync_copy(idx_hbm.at[sid], idx_v)
        loads = []
        for j in range(S_CHUNKS):
            c = sid * S_CHUNKS + j
            ic = c // chunks_per_i
            eo = pl.multiple_of((c % chunks_per_i) * CHUNK, CHUNK)
            loads.append(
                pltpu.async_copy(
                    msgs_hbm.at[ic, pl.ds(eo, CHUNK)],
                    msg_v.at[pl.ds(j * CHUNK, CHUNK)],
                    sems[j],
                )
            )
        zcp.wait()
        plsc.subcore_barrier()
        adds = []
        for j in range(S_CHUNKS):
            loads[j].wait()
            adds.append(
                pltpu.async_copy(
                    msg_v.at[pl.ds(j * CHUNK, CHUNK)],
                    acc.at[idx_v.at[j]],
                    sem_sc,
                    add=True,
                )
            )
        for a in adds:
            a.wait()
        plsc.subcore_barrier()
        out_rows = half // NS
        start = cid * half + sid * out_rows
        pltpu.sync_copy(
            acc.at[pl.ds(start, out_rows)], out_hbm.at[pl.ds(start, out_rows)]
        )

    return sk(msgs, idx3d, zeros)


BE = 2048


def _tc_dense_body(tr_ref, ids_ref, w2_ref, b2_ref, out_ref):
    ids = ids_ref[...]
    masks = []
    for i in range(ORDER):
        idv = lax.slice(ids, (0, i), (BE, i + 1))
        masks.append([idv == t for t in range(NUM_PARAMS)])
    transforms = [tr_ref[i] for i in range(ORDER)]
    facts = [
        transforms[1] * transforms[2],
        transforms[0] * transforms[2],
        transforms[0] * transforms[1],
    ]
    for i in range(ORDER):
        fcat = jnp.concatenate(
            [jnp.where(masks[i][t], facts[i], 0.0) for t in range(NUM_PARAMS)],
            axis=1,
        )
        msg = lax.dot_general(
            fcat, w2_ref[i], (((1,), (0,)), ((), ())),
            preferred_element_type=jnp.float32,
        )
        b2 = b2_ref[i]
        for t in range(NUM_PARAMS):
            msg = msg + jnp.where(masks[i][t], lax.slice(b2, (t, 0), (t + 1, LATENT)), 0.0)
        out_ref[i] = msg


def _tc_dense(transforms, ids, w2, b2, interpret=False):
    nb = E // BE
    return pl.pallas_call(
        _tc_dense_body,
        out_shape=jax.ShapeDtypeStruct((ORDER, E, LATENT), jnp.float32),
        grid=(nb,),
        in_specs=[
            pl.BlockSpec((ORDER, BE, RANK), lambda b: (0, b, 0)),
            pl.BlockSpec((BE, ORDER), lambda b: (b, 0)),
            pl.BlockSpec((ORDER, NUM_PARAMS * RANK, LATENT), lambda b: (0, 0, 0)),
            pl.BlockSpec((ORDER, NUM_PARAMS, LATENT), lambda b: (0, 0, 0)),
        ],
        out_specs=pl.BlockSpec((ORDER, BE, LATENT), lambda b: (0, b, 0)),
        interpret=interpret,
    )(transforms, ids, w2, b2)


def kernel(nodes, edges, edge_types, atoms, atom_edges, params, bias, ho_params, ho_bias):
    del atoms, atom_edges
    b1 = bias[:, 0, :]
    table = _tc_pre(nodes, params, b1)
    gidx = (edge_types.T.astype(jnp.int32) * N + edges.T.astype(jnp.int32))
    idx_g = gidx.reshape(NW, G_CHUNKS, CHUNK)
    idx_s = edges.T.reshape(NS, S_CHUNKS, CHUNK)
    tr = _sc_gather(table, idx_g)
    w2 = ho_params.reshape(ORDER, NUM_PARAMS * RANK, LATENT)
    b2 = ho_bias[:, :, 0, :]
    msgs = _tc_dense(tr, edge_types, w2, b2)
    zeros = jnp.zeros((N, LATENT), jnp.float32)
    return _sc_scatter_add(msgs, idx_s, zeros)


# speedup vs baseline: 52.4004x; 1.1561x over previous
import functools

import jax
import jax.numpy as jnp
from jax import lax
from jax.experimental import pallas as pl
from jax.experimental.pallas import tpu as pltpu
from jax.experimental.pallas import tpu_sc as plsc

N = 2048
E = 8192
ORDER = 3
LATENT = 64
RANK = 64
NUM_PARAMS = 4
PAD = 128

NC = 2
NS = 16
NW = NC * NS
R_TOTAL = ORDER * E
CHUNK = 128
R_PER_W = R_TOTAL // NW
G_CHUNKS = R_PER_W // CHUNK
R_PER_S = R_TOTAL // NS
S_CHUNKS = R_PER_S // CHUNK
WAVE = 3
N_WAVES = S_CHUNKS // WAVE

_SC_PARAMS = pltpu.CompilerParams(use_tc_tiling_on_sc=True)


def _tc_pre_body(nodes_ref, w1_ref, b1_ref, out_ref):
    x = nodes_ref[...]
    zpad = jnp.zeros((N, PAD - RANK), jnp.float32)
    for t in range(NUM_PARAMS):
        acc = lax.dot_general(
            x, w1_ref[t], (((1,), (0,)), ((), ())),
            preferred_element_type=jnp.float32,
        )
        b = b1_ref[...]
        r = jnp.maximum(acc + lax.slice(b, (t, 0), (t + 1, RANK)), 0.0)
        out_ref[pl.ds(t * N, N), :] = jnp.concatenate([r, zpad], axis=1)


def _tc_pre(nodes, w1, b1, interpret=False):
    return pl.pallas_call(
        _tc_pre_body,
        out_shape=jax.ShapeDtypeStruct((NUM_PARAMS * N, PAD), jnp.float32),
        interpret=interpret,
    )(nodes, w1, b1)


def _sc_gather(table, idx3d):
    mesh = plsc.VectorSubcoreMesh(core_axis_name="c", subcore_axis_name="s")
    chunks_per_i = E // CHUNK

    @functools.partial(
        pl.kernel,
        out_type=jax.ShapeDtypeStruct((ORDER, E, PAD), jnp.float32),
        mesh=mesh,
        scratch_types=[
            pltpu.VMEM((G_CHUNKS, CHUNK), jnp.int32),
            pltpu.VMEM((R_PER_W, PAD), jnp.float32),
            [pltpu.SemaphoreType.DMA] * G_CHUNKS,
            pltpu.SemaphoreType.DMA,
        ],
        compiler_params=_SC_PARAMS,
    )
    def gk(table_hbm, idx_hbm, out_hbm, idx_v, rows_v, sems, sem_wb):
        wid = lax.axis_index("s") * NC + lax.axis_index("c")
        pltpu.sync_copy(idx_hbm.at[wid], idx_v)
        copies = []
        for j in range(G_CHUNKS):
            copies.append(
                pltpu.async_copy(
                    table_hbm.at[idx_v.at[j]],
                    rows_v.at[pl.ds(j * CHUNK, CHUNK)],
                    sems[j],
                )
            )
        wbs = []
        for j in range(G_CHUNKS):
            c = wid * G_CHUNKS + j
            ic = c // chunks_per_i
            eo = pl.multiple_of((c % chunks_per_i) * CHUNK, CHUNK)
            copies[j].wait()
            wbs.append(
                pltpu.async_copy(
                    rows_v.at[pl.ds(j * CHUNK, CHUNK)],
                    out_hbm.at[ic, pl.ds(eo, CHUNK)],
                    sem_wb,
                )
            )
        for w in wbs:
            w.wait()

    return gk(table, idx3d)


def _sc_scatter_add(msgs, idx3d, zeros):
    mesh = plsc.VectorSubcoreMesh(core_axis_name="c", subcore_axis_name="s")
    half = N // NC
    chunks_per_i = E // CHUNK

    @functools.partial(
        pl.kernel,
        out_type=jax.ShapeDtypeStruct((N, PAD), jnp.float32),
        mesh=mesh,
        scratch_types=[
            pltpu.VMEM((S_CHUNKS, CHUNK), jnp.int32),
            [pltpu.VMEM((WAVE * CHUNK, PAD), jnp.float32)] * 2,
            pltpu.VMEM_SHARED((N, PAD), jnp.float32),
            pltpu.SemaphoreType.DMA,
            [pltpu.SemaphoreType.DMA] * 2,
            [pltpu.SemaphoreType.DMA] * 2,
        ],
        compiler_params=_SC_PARAMS,
    )
    def sk(msgs_hbm, idx_hbm, zeros_hbm, out_hbm, idx_v, bufs, acc, sem_z, sems_in, sems_sc):
        cid = lax.axis_index("c")
        sid = lax.axis_index("s")
        rows_per_tile = N // NS
        zcp = pltpu.async_copy(
            zeros_hbm.at[pl.ds(sid * rows_per_tile, rows_per_tile)],
            acc.at[pl.ds(sid * rows_per_tile, rows_per_tile)],
            sem_z,
        )
        pltpu.sync_copy(idx_hbm.at[sid], idx_v)

        def fire_loads(w, slot):
            ld = []
            for k in range(WAVE):
                c = sid * S_CHUNKS + w * WAVE + k
                ic = c // chunks_per_i
                eo = pl.multiple_of((c % chunks_per_i) * CHUNK, CHUNK)
                ld.append(
                    pltpu.async_copy(
                        msgs_hbm.at[ic, pl.ds(eo, CHUNK)],
                        bufs[slot].at[pl.ds(k * CHUNK, CHUNK)],
                        sems_in[slot],
                    )
                )
            return ld

        loads = [fire_loads(0, 0), fire_loads(1, 1)]
        zcp.wait()
        plsc.subcore_barrier()
        pending = [None, None]
        for w in range(N_WAVES):
            slot = w % 2
            for ld in loads[slot]:
                ld.wait()
            adds = []
            for k in range(WAVE):
                adds.append(
                    pltpu.async_copy(
                        bufs[slot].at[pl.ds(k * CHUNK, CHUNK)],
                        acc.at[idx_v.at[w * WAVE + k]],
                        sems_sc[slot],
                        add=True,
                    )
                )
            pending[slot] = adds
            if w + 2 < N_WAVES:
                for a in adds:
                    a.wait()
                pending[slot] = None
                loads[slot] = fire_loads(w + 2, slot)
        for adds in pending:
            if adds is not None:
                for a in adds:
                    a.wait()
        plsc.subcore_barrier()
        out_rows = half // NS
        start = cid * half + sid * out_rows
        pltpu.sync_copy(
            acc.at[pl.ds(start, out_rows)], out_hbm.at[pl.ds(start, out_rows)]
        )

    return sk(msgs, idx3d, zeros)


BE = 2048


def _tc_dense_body(tr_ref, ids_ref, w2_ref, b2_ref, out_ref):
    ids = ids_ref[...]
    masks = []
    for i in range(ORDER):
        idv = lax.slice(ids, (0, i), (BE, i + 1))
        masks.append([idv == t for t in range(NUM_PARAMS)])
    transforms = [
        lax.slice(tr_ref[i], (0, 0), (BE, RANK)) for i in range(ORDER)
    ]
    facts = [
        transforms[1] * transforms[2],
        transforms[0] * transforms[2],
        transforms[0] * transforms[1],
    ]
    zpad = jnp.zeros((BE, PAD - LATENT), jnp.float32)
    for i in range(ORDER):
        fcat = jnp.concatenate(
            [jnp.where(masks[i][t], facts[i], 0.0) for t in range(NUM_PARAMS)],
            axis=1,
        )
        msg = lax.dot_general(
            fcat, w2_ref[i], (((1,), (0,)), ((), ())),
            preferred_element_type=jnp.float32,
        )
        b2 = b2_ref[i]
        for t in range(NUM_PARAMS):
            msg = msg + jnp.where(masks[i][t], lax.slice(b2, (t, 0), (t + 1, LATENT)), 0.0)
        out_ref[i] = jnp.concatenate([msg, zpad], axis=1)


def _tc_dense(transforms, ids, w2, b2, interpret=False):
    nb = E // BE
    return pl.pallas_call(
        _tc_dense_body,
        out_shape=jax.ShapeDtypeStruct((ORDER, E, PAD), jnp.float32),
        grid=(nb,),
        in_specs=[
            pl.BlockSpec((ORDER, BE, PAD), lambda b: (0, b, 0)),
            pl.BlockSpec((BE, ORDER), lambda b: (b, 0)),
            pl.BlockSpec((ORDER, NUM_PARAMS * RANK, LATENT), lambda b: (0, 0, 0)),
            pl.BlockSpec((ORDER, NUM_PARAMS, LATENT), lambda b: (0, 0, 0)),
        ],
        out_specs=pl.BlockSpec((ORDER, BE, PAD), lambda b: (0, b, 0)),
        interpret=interpret,
    )(transforms, ids, w2, b2)


def kernel(nodes, edges, edge_types, atoms, atom_edges, params, bias, ho_params, ho_bias):
    del atoms, atom_edges
    b1 = bias[:, 0, :]
    table = _tc_pre(nodes, params, b1)
    gidx = (edge_types.T.astype(jnp.int32) * N + edges.T.astype(jnp.int32))
    idx_g = gidx.reshape(NW, G_CHUNKS, CHUNK)
    idx_s = edges.T.reshape(NS, S_CHUNKS, CHUNK)
    tr = _sc_gather(table, idx_g)
    w2 = ho_params.reshape(ORDER, NUM_PARAMS * RANK, LATENT)
    b2 = ho_bias[:, :, 0, :]
    msgs = _tc_dense(tr, edge_types, w2, b2)
    zeros = jnp.zeros((N, PAD), jnp.float32)
    out = _sc_scatter_add(msgs, idx_s, zeros)
    return lax.slice(out, (0, 0), (N, LATENT))


# speedup vs baseline: 56.3301x; 1.0750x over previous
import functools

import jax
import jax.numpy as jnp
from jax import lax
from jax.experimental import pallas as pl
from jax.experimental.pallas import tpu as pltpu
from jax.experimental.pallas import tpu_sc as plsc

N = 2048
E = 8192
ORDER = 3
LATENT = 64
RANK = 64
NUM_PARAMS = 4
PAD = 128

NC = 2
NS = 16
NW = NC * NS
R_TOTAL = ORDER * E
CHUNK = 128
R_PER_W = R_TOTAL // NW
G_CHUNKS = R_PER_W // CHUNK
R_PER_S = R_TOTAL // NW
S_CHUNKS = R_PER_S // CHUNK
WAVE = 3
N_WAVES = S_CHUNKS // WAVE

_SC_PARAMS = pltpu.CompilerParams(use_tc_tiling_on_sc=True)


def _tc_pre_body(nodes_ref, w1_ref, b1_ref, out_ref):
    x = nodes_ref[...]
    zpad = jnp.zeros((N, PAD - RANK), jnp.float32)
    for t in range(NUM_PARAMS):
        acc = lax.dot_general(
            x, w1_ref[t], (((1,), (0,)), ((), ())),
            preferred_element_type=jnp.float32,
        )
        b = b1_ref[...]
        r = jnp.maximum(acc + lax.slice(b, (t, 0), (t + 1, RANK)), 0.0)
        out_ref[pl.ds(t * N, N), :] = jnp.concatenate([r, zpad], axis=1)


def _tc_pre(nodes, w1, b1, interpret=False):
    return pl.pallas_call(
        _tc_pre_body,
        out_shape=jax.ShapeDtypeStruct((NUM_PARAMS * N, PAD), jnp.float32),
        interpret=interpret,
    )(nodes, w1, b1)


def _sc_gather(table, idx3d):
    mesh = plsc.VectorSubcoreMesh(core_axis_name="c", subcore_axis_name="s")
    chunks_per_i = E // CHUNK

    @functools.partial(
        pl.kernel,
        out_type=jax.ShapeDtypeStruct((ORDER, E, PAD), jnp.float32),
        mesh=mesh,
        scratch_types=[
            pltpu.VMEM((G_CHUNKS, CHUNK), jnp.int32),
            pltpu.VMEM((R_PER_W, PAD), jnp.float32),
            [pltpu.SemaphoreType.DMA] * G_CHUNKS,
            pltpu.SemaphoreType.DMA,
        ],
        compiler_params=_SC_PARAMS,
    )
    def gk(table_hbm, idx_hbm, out_hbm, idx_v, rows_v, sems, sem_wb):
        wid = lax.axis_index("s") * NC + lax.axis_index("c")
        pltpu.sync_copy(idx_hbm.at[wid], idx_v)
        copies = []
        for j in range(G_CHUNKS):
            copies.append(
                pltpu.async_copy(
                    table_hbm.at[idx_v.at[j]],
                    rows_v.at[pl.ds(j * CHUNK, CHUNK)],
                    sems[j],
                )
            )
        wbs = []
        for j in range(G_CHUNKS):
            c = wid * G_CHUNKS + j
            ic = c // chunks_per_i
            eo = pl.multiple_of((c % chunks_per_i) * CHUNK, CHUNK)
            copies[j].wait()
            wbs.append(
                pltpu.async_copy(
                    rows_v.at[pl.ds(j * CHUNK, CHUNK)],
                    out_hbm.at[ic, pl.ds(eo, CHUNK)],
                    sem_wb,
                )
            )
        for w in wbs:
            w.wait()

    return gk(table, idx3d)


def _sc_scatter_add(msgs, idx4d, zeros):
    mesh = plsc.VectorSubcoreMesh(core_axis_name="c", subcore_axis_name="s")
    chunks_per_i = E // CHUNK

    @functools.partial(
        pl.kernel,
        out_type=jax.ShapeDtypeStruct((NC, N, PAD), jnp.float32),
        mesh=mesh,
        scratch_types=[
            pltpu.VMEM((S_CHUNKS, CHUNK), jnp.int32),
            [pltpu.VMEM((WAVE * CHUNK, PAD), jnp.float32)] * 2,
            pltpu.VMEM_SHARED((N, PAD), jnp.float32),
            pltpu.SemaphoreType.DMA,
            [pltpu.SemaphoreType.DMA] * 2,
            [pltpu.SemaphoreType.DMA] * 2,
        ],
        compiler_params=_SC_PARAMS,
    )
    def sk(msgs_hbm, idx_hbm, zeros_hbm, out_hbm, idx_v, bufs, acc, sem_z, sems_in, sems_sc):
        cid = lax.axis_index("c")
        sid = lax.axis_index("s")
        rows_per_tile = N // NS
        zcp = pltpu.async_copy(
            zeros_hbm.at[pl.ds(sid * rows_per_tile, rows_per_tile)],
            acc.at[pl.ds(sid * rows_per_tile, rows_per_tile)],
            sem_z,
        )
        pltpu.sync_copy(idx_hbm.at[cid, sid], idx_v)

        def fire_loads(w, slot):
            ld = []
            for k in range(WAVE):
                c = (cid * NS + sid) * S_CHUNKS + w * WAVE + k
                ic = c // chunks_per_i
                eo = pl.multiple_of((c % chunks_per_i) * CHUNK, CHUNK)
                ld.append(
                    pltpu.async_copy(
                        msgs_hbm.at[ic, pl.ds(eo, CHUNK)],
                        bufs[slot].at[pl.ds(k * CHUNK, CHUNK)],
                        sems_in[slot],
                    )
                )
            return ld

        loads = [fire_loads(0, 0), fire_loads(1, 1)]
        zcp.wait()
        plsc.subcore_barrier()
        pending = [None, None]
        for w in range(N_WAVES):
            slot = w % 2
            for ld in loads[slot]:
                ld.wait()
            adds = []
            for k in range(WAVE):
                adds.append(
                    pltpu.async_copy(
                        bufs[slot].at[pl.ds(k * CHUNK, CHUNK)],
                        acc.at[idx_v.at[w * WAVE + k]],
                        sems_sc[slot],
                        add=True,
                    )
                )
            pending[slot] = adds
            if w + 2 < N_WAVES:
                for a in adds:
                    a.wait()
                pending[slot] = None
                loads[slot] = fire_loads(w + 2, slot)
        for adds in pending:
            if adds is not None:
                for a in adds:
                    a.wait()
        plsc.subcore_barrier()
        out_rows = N // NS
        start = sid * out_rows
        pltpu.sync_copy(
            acc.at[pl.ds(start, out_rows)],
            out_hbm.at[cid, pl.ds(start, out_rows)],
        )

    return sk(msgs, idx4d, zeros)


BE = 2048


def _tc_dense_body(tr_ref, ids_ref, w2_ref, b2_ref, out_ref):
    ids = ids_ref[...]
    masks = []
    for i in range(ORDER):
        idv = lax.slice(ids, (0, i), (BE, i + 1))
        masks.append([idv == t for t in range(NUM_PARAMS)])
    transforms = [
        lax.slice(tr_ref[i], (0, 0), (BE, RANK)) for i in range(ORDER)
    ]
    facts = [
        transforms[1] * transforms[2],
        transforms[0] * transforms[2],
        transforms[0] * transforms[1],
    ]
    zpad = jnp.zeros((BE, PAD - LATENT), jnp.float32)
    for i in range(ORDER):
        fcat = jnp.concatenate(
            [jnp.where(masks[i][t], facts[i], 0.0) for t in range(NUM_PARAMS)],
            axis=1,
        )
        msg = lax.dot_general(
            fcat, w2_ref[i], (((1,), (0,)), ((), ())),
            preferred_element_type=jnp.float32,
        )
        b2 = b2_ref[i]
        for t in range(NUM_PARAMS):
            msg = msg + jnp.where(masks[i][t], lax.slice(b2, (t, 0), (t + 1, LATENT)), 0.0)
        out_ref[i] = jnp.concatenate([msg, zpad], axis=1)


def _tc_dense(transforms, ids, w2, b2, interpret=False):
    nb = E // BE
    return pl.pallas_call(
        _tc_dense_body,
        out_shape=jax.ShapeDtypeStruct((ORDER, E, PAD), jnp.float32),
        grid=(nb,),
        in_specs=[
            pl.BlockSpec((ORDER, BE, PAD), lambda b: (0, b, 0)),
            pl.BlockSpec((BE, ORDER), lambda b: (b, 0)),
            pl.BlockSpec((ORDER, NUM_PARAMS * RANK, LATENT), lambda b: (0, 0, 0)),
            pl.BlockSpec((ORDER, NUM_PARAMS, LATENT), lambda b: (0, 0, 0)),
        ],
        out_specs=pl.BlockSpec((ORDER, BE, PAD), lambda b: (0, b, 0)),
        interpret=interpret,
    )(transforms, ids, w2, b2)


def kernel(nodes, edges, edge_types, atoms, atom_edges, params, bias, ho_params, ho_bias):
    del atoms, atom_edges
    b1 = bias[:, 0, :]
    table = _tc_pre(nodes, params, b1)
    gidx = (edge_types.T.astype(jnp.int32) * N + edges.T.astype(jnp.int32))
    idx_g = gidx.reshape(NW, G_CHUNKS, CHUNK)
    idx_s = edges.T.reshape(NC, NS, S_CHUNKS, CHUNK)
    tr = _sc_gather(table, idx_g)
    w2 = ho_params.reshape(ORDER, NUM_PARAMS * RANK, LATENT)
    b2 = ho_bias[:, :, 0, :]
    msgs = _tc_dense(tr, edge_types, w2, b2)
    zeros = jnp.zeros((N, PAD), jnp.float32)
    parts = _sc_scatter_add(msgs, idx_s, zeros)
    out = parts[0] + parts[1]
    return lax.slice(out, (0, 0), (N, LATENT))


# speedup vs baseline: 63.1164x; 1.1205x over previous
import functools

import jax
import jax.numpy as jnp
from jax import lax
from jax.experimental import pallas as pl
from jax.experimental.pallas import tpu as pltpu
from jax.experimental.pallas import tpu_sc as plsc

N = 2048
E = 8192
ORDER = 3
LATENT = 64
RANK = 64
NUM_PARAMS = 4
PAD = 128

NC = 2
NS = 16
NW = NC * NS
R_TOTAL = ORDER * E
CHUNK = 128
R_PER_W = R_TOTAL // NW
G_CHUNKS = R_PER_W // CHUNK
R_PER_S = R_TOTAL // NW
S_CHUNKS = R_PER_S // CHUNK
WAVE = 3
N_WAVES = S_CHUNKS // WAVE

_SC_PARAMS = pltpu.CompilerParams(use_tc_tiling_on_sc=True)


def _tc_pre_body(nodes_ref, w1_ref, b1_ref, out_ref):
    x = nodes_ref[...]
    zpad = jnp.zeros((N, PAD - RANK), jnp.float32)
    for t in range(NUM_PARAMS):
        acc = lax.dot_general(
            x, w1_ref[t], (((1,), (0,)), ((), ())),
            preferred_element_type=jnp.float32,
        )
        b = b1_ref[...]
        r = jnp.maximum(acc + lax.slice(b, (t, 0), (t + 1, RANK)), 0.0)
        out_ref[pl.ds(t * N, N), :] = jnp.concatenate([r, zpad], axis=1)


def _tc_pre(nodes, w1, b1, interpret=False):
    return pl.pallas_call(
        _tc_pre_body,
        out_shape=jax.ShapeDtypeStruct((NUM_PARAMS * N, PAD), jnp.float32),
        interpret=interpret,
    )(nodes, w1, b1)


def _sc_gather(table, idx3d):
    mesh = plsc.VectorSubcoreMesh(core_axis_name="c", subcore_axis_name="s")
    chunks_per_i = E // CHUNK

    @functools.partial(
        pl.kernel,
        out_type=jax.ShapeDtypeStruct((ORDER, E, PAD), jnp.float32),
        mesh=mesh,
        scratch_types=[
            pltpu.VMEM((G_CHUNKS, CHUNK), jnp.int32),
            pltpu.VMEM((R_PER_W, PAD), jnp.float32),
            [pltpu.SemaphoreType.DMA] * G_CHUNKS,
            pltpu.SemaphoreType.DMA,
        ],
        compiler_params=_SC_PARAMS,
    )
    def gk(table_hbm, idx_hbm, out_hbm, idx_v, rows_v, sems, sem_wb):
        wid = lax.axis_index("s") * NC + lax.axis_index("c")
        pltpu.sync_copy(idx_hbm.at[wid], idx_v)
        copies = []
        for j in range(G_CHUNKS):
            copies.append(
                pltpu.async_copy(
                    table_hbm.at[idx_v.at[j]],
                    rows_v.at[pl.ds(j * CHUNK, CHUNK)],
                    sems[j],
                )
            )
        wbs = []
        for j in range(G_CHUNKS):
            c = wid * G_CHUNKS + j
            ic = c // chunks_per_i
            eo = pl.multiple_of((c % chunks_per_i) * CHUNK, CHUNK)
            copies[j].wait()
            wbs.append(
                pltpu.async_copy(
                    rows_v.at[pl.ds(j * CHUNK, CHUNK)],
                    out_hbm.at[ic, pl.ds(eo, CHUNK)],
                    sem_wb,
                )
            )
        for w in wbs:
            w.wait()

    return gk(table, idx3d)


def _sc_scatter_add(msgs, idx4d, zeros):
    mesh = plsc.VectorSubcoreMesh(core_axis_name="c", subcore_axis_name="s")
    chunks_per_i = E // CHUNK

    @functools.partial(
        pl.kernel,
        out_type=jax.ShapeDtypeStruct((NC, N, PAD), jnp.float32),
        mesh=mesh,
        scratch_types=[
            pltpu.VMEM((S_CHUNKS, CHUNK), jnp.int32),
            [pltpu.VMEM((WAVE * CHUNK, PAD), jnp.float32)] * 2,
            pltpu.VMEM_SHARED((N, PAD), jnp.float32),
            pltpu.SemaphoreType.DMA,
            [pltpu.SemaphoreType.DMA] * 2,
            [pltpu.SemaphoreType.DMA] * 2,
        ],
        compiler_params=_SC_PARAMS,
    )
    def sk(msgs_hbm, idx_hbm, zeros_hbm, out_hbm, idx_v, bufs, acc, sem_z, sems_in, sems_sc):
        cid = lax.axis_index("c")
        sid = lax.axis_index("s")
        rows_per_tile = N // NS
        zcp = pltpu.async_copy(
            zeros_hbm.at[pl.ds(sid * rows_per_tile, rows_per_tile)],
            acc.at[pl.ds(sid * rows_per_tile, rows_per_tile)],
            sem_z,
        )
        pltpu.sync_copy(idx_hbm.at[cid, sid], idx_v)

        def fire_loads(w, slot):
            ld = []
            for k in range(WAVE):
                c = (cid * NS + sid) * S_CHUNKS + w * WAVE + k
                ic = c // chunks_per_i
                eo = pl.multiple_of((c % chunks_per_i) * CHUNK, CHUNK)
                ld.append(
                    pltpu.async_copy(
                        msgs_hbm.at[ic, pl.ds(eo, CHUNK)],
                        bufs[slot].at[pl.ds(k * CHUNK, CHUNK)],
                        sems_in[slot],
                    )
                )
            return ld

        loads = [fire_loads(0, 0), fire_loads(1, 1)]
        zcp.wait()
        plsc.subcore_barrier()
        pending = [None, None]
        for w in range(N_WAVES):
            slot = w % 2
            for ld in loads[slot]:
                ld.wait()
            adds = []
            for k in range(WAVE):
                adds.append(
                    pltpu.async_copy(
                        bufs[slot].at[pl.ds(k * CHUNK, CHUNK)],
                        acc.at[idx_v.at[w * WAVE + k]],
                        sems_sc[slot],
                        add=True,
                    )
                )
            pending[slot] = adds
            if w + 2 < N_WAVES:
                for a in adds:
                    a.wait()
                pending[slot] = None
                loads[slot] = fire_loads(w + 2, slot)
        for adds in pending:
            if adds is not None:
                for a in adds:
                    a.wait()
        plsc.subcore_barrier()
        out_rows = N // NS
        start = sid * out_rows
        pltpu.sync_copy(
            acc.at[pl.ds(start, out_rows)],
            out_hbm.at[cid, pl.ds(start, out_rows)],
        )

    return sk(msgs, idx4d, zeros)


BE = 2048


def _tc_dense_body(tr_ref, ids_ref, w2_ref, b2_ref, out_ref):
    ids = ids_ref[...]
    masks = []
    for i in range(ORDER):
        idv = lax.slice(ids, (0, i), (BE, i + 1))
        masks.append([idv == t for t in range(NUM_PARAMS - 1)])
    transforms = [
        lax.slice(tr_ref[i], (0, 0), (BE, RANK)) for i in range(ORDER)
    ]
    facts = [
        transforms[1] * transforms[2],
        transforms[0] * transforms[2],
        transforms[0] * transforms[1],
    ]
    zpad = jnp.zeros((BE, PAD - LATENT), jnp.float32)
    for i in range(ORDER):
        w2 = w2_ref[i]
        b2 = b2_ref[i]
        ys = []
        for t in range(NUM_PARAMS):
            y = lax.dot_general(
                facts[i], lax.slice(w2, (t * RANK, 0), ((t + 1) * RANK, LATENT)),
                (((1,), (0,)), ((), ())),
                preferred_element_type=jnp.float32,
            )
            ys.append(y + lax.slice(b2, (t, 0), (t + 1, LATENT)))
        m = masks[i]
        msg = jnp.where(
            m[0], ys[0], jnp.where(m[1], ys[1], jnp.where(m[2], ys[2], ys[3]))
        )
        out_ref[i] = jnp.concatenate([msg, zpad], axis=1)


def _tc_dense(transforms, ids, w2, b2, interpret=False):
    nb = E // BE
    return pl.pallas_call(
        _tc_dense_body,
        out_shape=jax.ShapeDtypeStruct((ORDER, E, PAD), jnp.float32),
        grid=(nb,),
        in_specs=[
            pl.BlockSpec((ORDER, BE, PAD), lambda b: (0, b, 0)),
            pl.BlockSpec((BE, ORDER), lambda b: (b, 0)),
            pl.BlockSpec((ORDER, NUM_PARAMS * RANK, LATENT), lambda b: (0, 0, 0)),
            pl.BlockSpec((ORDER, NUM_PARAMS, LATENT), lambda b: (0, 0, 0)),
        ],
        out_specs=pl.BlockSpec((ORDER, BE, PAD), lambda b: (0, b, 0)),
        interpret=interpret,
    )(transforms, ids, w2, b2)


def kernel(nodes, edges, edge_types, atoms, atom_edges, params, bias, ho_params, ho_bias):
    del atoms, atom_edges
    b1 = bias[:, 0, :]
    table = _tc_pre(nodes, params, b1)
    gidx = (edge_types.T.astype(jnp.int32) * N + edges.T.astype(jnp.int32))
    idx_g = gidx.reshape(NW, G_CHUNKS, CHUNK)
    idx_s = edges.T.reshape(NC, NS, S_CHUNKS, CHUNK)
    tr = _sc_gather(table, idx_g)
    w2 = ho_params.reshape(ORDER, NUM_PARAMS * RANK, LATENT)
    b2 = ho_bias[:, :, 0, :]
    msgs = _tc_dense(tr, edge_types, w2, b2)
    zeros = jnp.zeros((N, PAD), jnp.float32)
    parts = _sc_scatter_add(msgs, idx_s, zeros)
    out = parts[0] + parts[1]
    return lax.slice(out, (0, 0), (N, LATENT))
